# baseline (speedup 1.0000x reference)
import functools

import jax
import jax.numpy as jnp
from jax import lax
from jax.experimental import pallas as pl
from jax.experimental.pallas import tpu as pltpu
from jax.experimental.pallas import tpu_sc as plsc

NC, NS = 2, 16
NW = NC * NS
CHUNK = 128
NSEG = 16384


def _elu(x):
    return jnp.where(x > 0, x, jnp.exp(x) - 1.0)


def _mesh():
    return plsc.VectorSubcoreMesh(core_axis_name="c", subcore_axis_name="s",
                                  num_cores=NC, num_subcores=NS)


_SC_PARAMS = pltpu.CompilerParams(use_tc_tiling_on_sc=False)


def _sc_gather(table, idx2):
    m = idx2.shape[0] * 128
    rows_pw = m // NW
    nchunks = rows_pw // CHUNK

    k_b = 8

    @functools.partial(
        pl.kernel,
        out_type=jax.ShapeDtypeStruct((m, 32), jnp.float32),
        mesh=_mesh(),
        scratch_types=[
            pltpu.VMEM((k_b, CHUNK), jnp.int32),
            pltpu.VMEM((k_b, CHUNK, 32), jnp.float32),
            pltpu.SemaphoreType.DMA,
            pltpu.SemaphoreType.DMA,
            pltpu.SemaphoreType.DMA,
        ],
        compiler_params=_SC_PARAMS,
    )
    def k(table_hbm, idx_hbm, out_hbm, idx_v, rows_v, sem_i, sem_g, sem_o):
        wid = lax.axis_index("s") * NC + lax.axis_index("c")
        base = wid * rows_pw

        def body(i, carry):
            off = base + i * (k_b * CHUNK)
            pltpu.async_copy(idx_hbm.at[pl.ds(off // 128, k_b * CHUNK // 128)],
                             idx_v, sem_i).wait()
            ds = [pltpu.async_copy(table_hbm.at[idx_v.at[b]], rows_v.at[b], sem_g)
                  for b in range(k_b)]
            for dsc in ds:
                dsc.wait()
            ds = [pltpu.async_copy(rows_v.at[b], out_hbm.at[pl.ds(off + b * CHUNK, CHUNK)],
                                   sem_o) for b in range(k_b)]
            for dsc in ds:
                dsc.wait()
            return carry

        lax.fori_loop(0, nchunks // k_b, body, 0)

    return k(table, idx2)


def _sc_segsum1(y, seg2):
    m = y.shape[0]
    rows_pw1 = m // NW
    rows_pt = NSEG // NS

    k_b = 8

    @functools.partial(
        pl.kernel,
        out_type=(
            jax.ShapeDtypeStruct((NC, NSEG, 32), jnp.float32),
            jax.ShapeDtypeStruct((NC, NSEG, 16), jnp.float32),
        ),
        mesh=_mesh(),
        scratch_types=[
            pltpu.VMEM((k_b, CHUNK), jnp.int32),
            pltpu.VMEM((k_b, CHUNK, 32), jnp.float32),
            pltpu.VMEM((CHUNK, 32), jnp.float32),
            pltpu.VMEM((CHUNK, 16), jnp.float32),
            pltpu.VMEM((CHUNK, 16), jnp.float32),
            pltpu.VMEM_SHARED((NSEG, 32), jnp.float32),
            pltpu.VMEM_SHARED((NSEG, 16), jnp.float32),
            pltpu.SemaphoreType.DMA,
            pltpu.SemaphoreType.DMA,
            pltpu.SemaphoreType.DMA,
        ],
        compiler_params=_SC_PARAMS,
    )
    def k(y_hbm, seg_hbm, o1, oc1, idx_v, rows_v, z32_v, z16_v, one16_v,
          acc1, accc, sem_i, sem_r, sem_s):
        cid = lax.axis_index("c")
        scid = lax.axis_index("s")
        wid = scid * NC + cid

        z = jnp.zeros((16,), jnp.float32)
        o = jnp.ones((16,), jnp.float32)
        for row in range(CHUNK):
            z32_v[row, 0:16] = z
            z32_v[row, 16:32] = z
            z16_v[row, 0:16] = z
            one16_v[row, 0:16] = o

        base_t = scid * rows_pt
        for j in range(rows_pt // CHUNK):
            pltpu.sync_copy(z32_v, acc1.at[pl.ds(base_t + j * CHUNK, CHUNK)])
            pltpu.sync_copy(z16_v, accc.at[pl.ds(base_t + j * CHUNK, CHUNK)])
        plsc.subcore_barrier()

        base1 = wid * rows_pw1

        def b1(i, carry):
            off = base1 + i * (k_b * CHUNK)
            ds = ([pltpu.async_copy(seg_hbm.at[pl.ds(off // 128, k_b * CHUNK // 128)],
                                    idx_v, sem_i)]
                  + [pltpu.async_copy(y_hbm.at[pl.ds(off + b * CHUNK, CHUNK)],
                                      rows_v.at[b], sem_r) for b in range(k_b)])
            for dsc in ds:
                dsc.wait()
            ds = ([pltpu.async_copy(rows_v.at[b], acc1.at[idx_v.at[b]], sem_s, add=True)
                   for b in range(k_b)]
                  + [pltpu.async_copy(one16_v, accc.at[idx_v.at[b]], sem_s, add=True)
                     for b in range(k_b)])
            for dsc in ds:
                dsc.wait()
            return carry

        lax.fori_loop(0, rows_pw1 // (k_b * CHUNK), b1, 0)
        plsc.subcore_barrier()

        for j in range(rows_pt // CHUNK):
            r0 = base_t + j * CHUNK
            pltpu.sync_copy(acc1.at[pl.ds(r0, CHUNK)], o1.at[cid, pl.ds(r0, CHUNK)])
            pltpu.sync_copy(accc.at[pl.ds(r0, CHUNK)], oc1.at[cid, pl.ds(r0, CHUNK)])

    return k(y, seg2)


def _sc_segsum2(feat, sid2):
    r = feat.shape[0]
    rows_pw2 = r // NW
    rows_pt = NSEG // NS

    k_b = 8

    @functools.partial(
        pl.kernel,
        out_type=(
            jax.ShapeDtypeStruct((NC, NSEG, 32), jnp.float32),
            jax.ShapeDtypeStruct((NC, NSEG, 16), jnp.float32),
        ),
        mesh=_mesh(),
        scratch_types=[
            pltpu.VMEM((k_b, CHUNK), jnp.int32),
            pltpu.VMEM((k_b, CHUNK, 32), jnp.float32),
            pltpu.VMEM((CHUNK, 32), jnp.float32),
            pltpu.VMEM((CHUNK, 16), jnp.float32),
            pltpu.VMEM((CHUNK, 16), jnp.float32),
            pltpu.VMEM_SHARED((NSEG, 32), jnp.float32),
            pltpu.VMEM_SHARED((NSEG, 16), jnp.float32),
            pltpu.SemaphoreType.DMA,
            pltpu.SemaphoreType.DMA,
            pltpu.SemaphoreType.DMA,
        ],
        compiler_params=_SC_PARAMS,
    )
    def k(feat_hbm, sid_hbm, o2, oc,
          idx_v, rows_v, z32_v, z16_v, one16_v, acc2, accc, sem_i, sem_r, sem_s):
        cid = lax.axis_index("c")
        scid = lax.axis_index("s")
        wid = scid * NC + cid

        z = jnp.zeros((16,), jnp.float32)
        o = jnp.ones((16,), jnp.float32)
        for row in range(CHUNK):
            z32_v[row, 0:16] = z
            z32_v[row, 16:32] = z
            z16_v[row, 0:16] = z
            one16_v[row, 0:16] = o

        base_t = scid * rows_pt
        for j in range(rows_pt // CHUNK):
            r0 = base_t + j * CHUNK
            pltpu.sync_copy(z32_v, acc2.at[pl.ds(r0, CHUNK)])
            pltpu.sync_copy(z16_v, accc.at[pl.ds(r0, CHUNK)])
        plsc.subcore_barrier()

        base2 = wid * rows_pw2

        def b2(i, carry):
            off = base2 + i * (k_b * CHUNK)
            ds = ([pltpu.async_copy(sid_hbm.at[pl.ds(off // 128, k_b * CHUNK // 128)],
                                    idx_v, sem_i)]
                  + [pltpu.async_copy(feat_hbm.at[pl.ds(off + b * CHUNK, CHUNK)],
                                      rows_v.at[b], sem_r) for b in range(k_b)])
            for dsc in ds:
                dsc.wait()
            ds = ([pltpu.async_copy(rows_v.at[b], acc2.at[idx_v.at[b]], sem_s, add=True)
                   for b in range(k_b)]
                  + [pltpu.async_copy(one16_v, accc.at[idx_v.at[b]], sem_s, add=True)
                     for b in range(k_b)])
            for dsc in ds:
                dsc.wait()
            return carry

        lax.fori_loop(0, rows_pw2 // (k_b * CHUNK), b2, 0)
        plsc.subcore_barrier()

        for j in range(rows_pt // CHUNK):
            r0 = base_t + j * CHUNK
            pltpu.sync_copy(acc2.at[pl.ds(r0, CHUNK)], o2.at[cid, pl.ds(r0, CHUNK)])
            pltpu.sync_copy(accc.at[pl.ds(r0, CHUNK)], oc.at[cid, pl.ds(r0, CHUNK)])

    return k(feat, sid2)


def _tc_row_mlps_packed(snf128, w1d, b1t, w2d, b2t):
    mp = snf128.shape[0]
    br = 1024
    full = lambda: pl.BlockSpec((128, 128), lambda i: (0, 0))
    bias = lambda: pl.BlockSpec((1, 128), lambda i: (0, 0))
    rowb = lambda: pl.BlockSpec((br, 128), lambda i: (i, 0))

    def body(s_ref, w1, b1, w2, b2, o_ref):
        x = s_ref[...]
        h = _elu(jnp.dot(x, w1[...], preferred_element_type=jnp.float32) + b1[...])
        o_ref[...] = jnp.dot(h, w2[...], preferred_element_type=jnp.float32) + b2[...]

    return pl.pallas_call(
        body,
        grid=(mp // br,),
        in_specs=[rowb(), full(), bias(), full(), bias()],
        out_specs=rowb(),
        out_shape=jax.ShapeDtypeStruct((mp, 128), jnp.float32),
    )(snf128, w1d, b1t, w2d, b2t)


def _blkdiag4(w):
    z = jnp.zeros((32, 32), w.dtype)
    return jnp.block([[w, z, z, z], [z, w, z, z], [z, z, w, z], [z, z, z, w]])


BR = 512


def _tc_row_mlps_segsum(snf, g, seg, nn_W1, nn_b1, nn_W2, nn_b2,
                        gm1_W1, gm1_b1, gm1_W2, gm1_b2):
    m = snf.shape[0]
    nb = m // BR
    firsts = seg[::BR]
    seg3 = seg.reshape(nb, 1, BR)
    full = lambda: pl.BlockSpec((32, 32), lambda i, f: (0, 0))
    bias = lambda: pl.BlockSpec((1, 32), lambda i, f: (0, 0))
    rowb = lambda: pl.BlockSpec((BR, 32), lambda i, f: (i, 0))

    def body(f_ref, s_ref, g_ref, seg_ref, w1, b1, w2, b2, v1, c1, v2, c2, o_ref):
        i = pl.program_id(0)

        @pl.when(i == 0)
        def _init():
            o_ref[...] = jnp.zeros((NSEG + BR, 32), jnp.float32)

        x = s_ref[...]
        h = _elu(jnp.dot(x, w1[...], preferred_element_type=jnp.float32) + b1[...])
        h = jnp.dot(h, w2[...], preferred_element_type=jnp.float32) + b2[...] + g_ref[...]
        h2 = _elu(jnp.dot(h, v1[...], preferred_element_type=jnp.float32) + c1[...])
        y = jnp.dot(h2, v2[...], preferred_element_type=jnp.float32) + c2[...]

        first = f_ref[i]
        rel = seg_ref[0] - first
        pt = (jax.lax.broadcasted_iota(jnp.int32, (BR, BR), 0) == rel
              ).astype(jnp.bfloat16)
        yh = y.astype(jnp.bfloat16)
        yl = (y - yh.astype(jnp.float32)).astype(jnp.bfloat16)
        sblk = (jnp.dot(pt, yh, preferred_element_type=jnp.float32)
                + jnp.dot(pt, yl, preferred_element_type=jnp.float32))
        o_ref[pl.ds(first, BR), :] += sblk

    acc = pl.pallas_call(
        body,
        grid_spec=pltpu.PrefetchScalarGridSpec(
            num_scalar_prefetch=1,
            grid=(nb,),
            in_specs=[rowb(), rowb(), pl.BlockSpec((1, 1, BR), lambda i, f: (i, 0, 0)),
                      full(), bias(), full(), bias(), full(), bias(), full(), bias()],
            out_specs=pl.BlockSpec((NSEG + BR, 32), lambda i, f: (0, 0)),
        ),
        out_shape=jax.ShapeDtypeStruct((NSEG + BR, 32), jnp.float32),
    )(firsts, snf, g, seg3, nn_W1, nn_b1.reshape(1, 32), nn_W2, nn_b2.reshape(1, 32),
      gm1_W1, gm1_b1.reshape(1, 32), gm1_W2, gm1_b2.reshape(1, 32))
    return acc[:NSEG]


def _tc_head(s1zp, c1p, s2p, cp, v2, c2, gm2_W1, gm2_b1, gm2_W2, gm2_b2,
             fm_W1, fm_b1, fm_W2, fm_b2, oW1f, oW1g, ob1, oW2, ob2):
    br = 2048
    n_label = oW2.shape[1]
    full = lambda: pl.BlockSpec((32, 32), lambda i: (0, 0))
    bias = lambda: pl.BlockSpec((1, 32), lambda i: (0, 0))

    def body(s1z_ref, c1_ref, s2_ref, c_ref, v2_ref, c2_ref, g1, gb1, g2, gb2,
             f1, fb1, f2, fb2, w1f, w1g, b1, w2, b2, o_ref):
        s1z = s1z_ref[0] + s1z_ref[1]
        n1 = (c1_ref[0] + c1_ref[1])[:, 0:1]
        s1 = (jnp.dot(s1z, v2_ref[...], preferred_element_type=jnp.float32,
                      precision=jax.lax.Precision.HIGHEST)
              + n1 * c2_ref[...])
        s2 = s2_ref[0] + s2_ref[1]
        cnt = c_ref[0] + c_ref[1]
        cnt1 = jnp.clip(cnt[:, 0:1], 1.0, None)
        og = _elu(jnp.dot(s1, g1[...], preferred_element_type=jnp.float32) + gb1[...])
        og = jnp.dot(og, g2[...], preferred_element_type=jnp.float32) + gb2[...]
        mean = s2 / cnt1
        of = _elu(jnp.dot(mean, f1[...], preferred_element_type=jnp.float32) + fb1[...])
        of = jnp.dot(of, f2[...], preferred_element_type=jnp.float32) + fb2[...]
        h = _elu(jnp.dot(of, w1f[...], preferred_element_type=jnp.float32)
                 + jnp.dot(og, w1g[...], preferred_element_type=jnp.float32) + b1[...])
        o_ref[...] = jnp.dot(h, w2[...], preferred_element_type=jnp.float32) + b2[...]

    return pl.pallas_call(
        body,
        grid=(NSEG // br,),
        in_specs=[
            pl.BlockSpec((2, br, 32), lambda i: (0, i, 0)),
            pl.BlockSpec((2, br, 16), lambda i: (0, i, 0)),
            pl.BlockSpec((2, br, 32), lambda i: (0, i, 0)),
            pl.BlockSpec((2, br, 16), lambda i: (0, i, 0)),
            full(), bias(),
            full(), bias(), full(), bias(), full(), bias(), full(), bias(),
            full(), full(), bias(),
            pl.BlockSpec((32, n_label), lambda i: (0, 0)),
            pl.BlockSpec((1, n_label), lambda i: (0, 0)),
        ],
        out_specs=pl.BlockSpec((br, n_label), lambda i: (i, 0)),
        out_shape=jax.ShapeDtypeStruct((NSEG, n_label), jnp.float32),
    )(s1zp, c1p, s2p, cp, v2, c2.reshape(1, 32),
      gm2_W1, gm2_b1.reshape(1, 32), gm2_W2, gm2_b2.reshape(1, 32),
      fm_W1, fm_b1.reshape(1, 32), fm_W2, fm_b2.reshape(1, 32),
      oW1f, oW1g, ob1.reshape(1, 32), oW2, ob2.reshape(1, n_label))


def kernel(graph_out, sample_node_id, sample_node_feature, sample_id, sample_feature,
           nn_W1, nn_b1, nn_W2, nn_b2, gm1_W1, gm1_b1, gm1_W2, gm1_b2,
           gm2_W1, gm2_b1, gm2_W2, gm2_b2, fm_W1, fm_b1, fm_W2, fm_b2,
           out_W1, out_b1, out_W2, out_b2):
    seg1 = sample_node_id[:, 0]
    nid = sample_node_id[:, 1]

    m = sample_node_feature.shape[0]
    r = sample_feature.shape[0]
    hi = jax.lax.Precision.HIGHEST
    table2 = jnp.dot(graph_out, gm1_W1, preferred_element_type=jnp.float32,
                     precision=hi)
    w2v1 = jnp.dot(nn_W2, gm1_W1, preferred_element_type=jnp.float32, precision=hi)
    bias2 = jnp.dot(nn_b2, gm1_W1, preferred_element_type=jnp.float32,
                    precision=hi) + gm1_b1

    g2 = _sc_gather(table2, nid.reshape(m // 128, 128))
    s2p, cp = _sc_segsum2(sample_feature, sample_id.reshape(r // 128, 128))
    t128 = _tc_row_mlps_packed(
        sample_node_feature.reshape(m // 4, 128),
        _blkdiag4(nn_W1), jnp.tile(nn_b1, 4).reshape(1, 128),
        _blkdiag4(w2v1), jnp.tile(bias2, 4).reshape(1, 128))
    z128 = _elu(t128 + g2.reshape(m // 4, 128))
    s1zp, c1p = _sc_segsum1(z128.reshape(m, 32), seg1.reshape(m // 128, 128))
    return _tc_head(s1zp, c1p, s2p, cp, gm1_W2, gm1_b2,
                    gm2_W1, gm2_b1, gm2_W2, gm2_b2,
                    fm_W1, fm_b1, fm_W2, fm_b2,
                    out_W1[:32], out_W1[32:], out_b1, out_W2, out_b2)


# speedup vs baseline: 1.1600x; 1.1600x over previous
import functools

import jax
import jax.numpy as jnp
from jax import lax
from jax.experimental import pallas as pl
from jax.experimental.pallas import tpu as pltpu
from jax.experimental.pallas import tpu_sc as plsc

NC, NS = 2, 16
NW = NC * NS
CHUNK = 128
NSEG = 16384


def _elu(x):
    return jnp.where(x > 0, x, jnp.exp(x) - 1.0)


def _mesh():
    return plsc.VectorSubcoreMesh(core_axis_name="c", subcore_axis_name="s",
                                  num_cores=NC, num_subcores=NS)


_SC_PARAMS = pltpu.CompilerParams(use_tc_tiling_on_sc=False)


def _sc_gather(table, idx2):
    m = idx2.shape[0] * 128
    rows_pw = m // NW
    nchunks = rows_pw // CHUNK

    k_b = 8

    @functools.partial(
        pl.kernel,
        out_type=jax.ShapeDtypeStruct((m // 4, 128), jnp.float32),
        mesh=_mesh(),
        scratch_types=[
            pltpu.VMEM((k_b, CHUNK), jnp.int32),
            pltpu.VMEM((k_b, CHUNK, 32), jnp.float32),
            pltpu.VMEM((k_b * CHUNK // 4, 128), jnp.float32),
            pltpu.SemaphoreType.DMA,
            pltpu.SemaphoreType.DMA,
            pltpu.SemaphoreType.DMA,
        ],
        compiler_params=_SC_PARAMS,
    )
    def k(table_hbm, idx_hbm, out_hbm, idx_v, rows_v, rows128_v, sem_i, sem_g, sem_o):
        wid = lax.axis_index("s") * NC + lax.axis_index("c")
        base = wid * rows_pw

        def body(i, carry):
            off = base + i * (k_b * CHUNK)
            pltpu.async_copy(idx_hbm.at[pl.ds(off // 128, k_b * CHUNK // 128)],
                             idx_v, sem_i).wait()
            ds = [pltpu.async_copy(table_hbm.at[idx_v.at[b]], rows_v.at[b], sem_g)
                  for b in range(k_b)]
            for dsc in ds:
                dsc.wait()

            def repack(q, c2):
                b = q // (CHUNK // 4)
                rr = 4 * (q % (CHUNK // 4))
                for j in range(4):
                    rows128_v[q, 32 * j:32 * j + 16] = rows_v[b, rr + j, 0:16]
                    rows128_v[q, 32 * j + 16:32 * j + 32] = rows_v[b, rr + j, 16:32]
                return c2

            lax.fori_loop(0, k_b * CHUNK // 4, repack, 0, unroll=8)
            pltpu.async_copy(rows128_v, out_hbm.at[pl.ds(off // 4, k_b * CHUNK // 4)],
                             sem_o).wait()
            return carry

        lax.fori_loop(0, nchunks // k_b, body, 0)

    return k(table, idx2)


def _sc_segsum1(y, seg2):
    m = y.shape[0]
    rows_pw1 = m // NW
    rows_pt = NSEG // NS

    k_b = 8

    @functools.partial(
        pl.kernel,
        out_type=jax.ShapeDtypeStruct((NC, NSEG, 32), jnp.float32),
        mesh=_mesh(),
        scratch_types=[
            pltpu.VMEM((k_b, CHUNK), jnp.int32),
            pltpu.VMEM((k_b, CHUNK, 32), jnp.float32),
            pltpu.VMEM((CHUNK, 32), jnp.float32),
            pltpu.VMEM_SHARED((NSEG, 32), jnp.float32),
            pltpu.SemaphoreType.DMA,
            pltpu.SemaphoreType.DMA,
            pltpu.SemaphoreType.DMA,
        ],
        compiler_params=_SC_PARAMS,
    )
    def k(y_hbm, seg_hbm, o1, idx_v, rows_v, z32_v, acc1, sem_i, sem_r, sem_s):
        cid = lax.axis_index("c")
        scid = lax.axis_index("s")
        wid = scid * NC + cid

        z = jnp.zeros((16,), jnp.float32)
        for row in range(CHUNK):
            z32_v[row, 0:16] = z
            z32_v[row, 16:32] = z

        base_t = scid * rows_pt
        for j in range(rows_pt // CHUNK):
            pltpu.sync_copy(z32_v, acc1.at[pl.ds(base_t + j * CHUNK, CHUNK)])
        plsc.subcore_barrier()

        base1 = wid * rows_pw1

        def b1(i, carry):
            off = base1 + i * (k_b * CHUNK)
            ds = ([pltpu.async_copy(seg_hbm.at[pl.ds(off // 128, k_b * CHUNK // 128)],
                                    idx_v, sem_i)]
                  + [pltpu.async_copy(y_hbm.at[pl.ds(off + b * CHUNK, CHUNK)],
                                      rows_v.at[b], sem_r) for b in range(k_b)])
            for dsc in ds:
                dsc.wait()
            ds = [pltpu.async_copy(rows_v.at[b], acc1.at[idx_v.at[b]], sem_s, add=True)
                  for b in range(k_b)]
            for dsc in ds:
                dsc.wait()
            return carry

        lax.fori_loop(0, rows_pw1 // (k_b * CHUNK), b1, 0)
        plsc.subcore_barrier()

        for j in range(rows_pt // CHUNK):
            r0 = base_t + j * CHUNK
            pltpu.sync_copy(acc1.at[pl.ds(r0, CHUNK)], o1.at[cid, pl.ds(r0, CHUNK)])

    return k(y, seg2)


def _sc_segsum2(feat, sid2):
    r = feat.shape[0]
    rows_pw2 = r // NW
    rows_pt = NSEG // NS

    k_b = 8

    @functools.partial(
        pl.kernel,
        out_type=(
            jax.ShapeDtypeStruct((NC, NSEG, 32), jnp.float32),
            jax.ShapeDtypeStruct((NC, NSEG, 16), jnp.float32),
        ),
        mesh=_mesh(),
        scratch_types=[
            pltpu.VMEM((k_b, CHUNK), jnp.int32),
            pltpu.VMEM((k_b, CHUNK, 32), jnp.float32),
            pltpu.VMEM((CHUNK, 32), jnp.float32),
            pltpu.VMEM((CHUNK, 16), jnp.float32),
            pltpu.VMEM((CHUNK, 16), jnp.float32),
            pltpu.VMEM_SHARED((NSEG, 32), jnp.float32),
            pltpu.VMEM_SHARED((NSEG, 16), jnp.float32),
            pltpu.SemaphoreType.DMA,
            pltpu.SemaphoreType.DMA,
            pltpu.SemaphoreType.DMA,
        ],
        compiler_params=_SC_PARAMS,
    )
    def k(feat_hbm, sid_hbm, o2, oc,
          idx_v, rows_v, z32_v, z16_v, one16_v, acc2, accc, sem_i, sem_r, sem_s):
        cid = lax.axis_index("c")
        scid = lax.axis_index("s")
        wid = scid * NC + cid

        z = jnp.zeros((16,), jnp.float32)
        o = jnp.ones((16,), jnp.float32)
        for row in range(CHUNK):
            z32_v[row, 0:16] = z
            z32_v[row, 16:32] = z
            z16_v[row, 0:16] = z
            one16_v[row, 0:16] = o

        base_t = scid * rows_pt
        for j in range(rows_pt // CHUNK):
            r0 = base_t + j * CHUNK
            pltpu.sync_copy(z32_v, acc2.at[pl.ds(r0, CHUNK)])
            pltpu.sync_copy(z16_v, accc.at[pl.ds(r0, CHUNK)])
        plsc.subcore_barrier()

        base2 = wid * rows_pw2

        def b2(i, carry):
            off = base2 + i * (k_b * CHUNK)
            ds = ([pltpu.async_copy(sid_hbm.at[pl.ds(off // 128, k_b * CHUNK // 128)],
                                    idx_v, sem_i)]
                  + [pltpu.async_copy(feat_hbm.at[pl.ds(off + b * CHUNK, CHUNK)],
                                      rows_v.at[b], sem_r) for b in range(k_b)])
            for dsc in ds:
                dsc.wait()
            ds = ([pltpu.async_copy(rows_v.at[b], acc2.at[idx_v.at[b]], sem_s, add=True)
                   for b in range(k_b)]
                  + [pltpu.async_copy(one16_v, accc.at[idx_v.at[b]], sem_s, add=True)
                     for b in range(k_b)])
            for dsc in ds:
                dsc.wait()
            return carry

        lax.fori_loop(0, rows_pw2 // (k_b * CHUNK), b2, 0)
        plsc.subcore_barrier()

        for j in range(rows_pt // CHUNK):
            r0 = base_t + j * CHUNK
            pltpu.sync_copy(acc2.at[pl.ds(r0, CHUNK)], o2.at[cid, pl.ds(r0, CHUNK)])
            pltpu.sync_copy(accc.at[pl.ds(r0, CHUNK)], oc.at[cid, pl.ds(r0, CHUNK)])

    return k(feat, sid2)


def _tc_row_mlps_packed(snf128, g128, w1d, b1t, w2d, b2t, v1d, c1t, v2d, c2t):
    mp = snf128.shape[0]
    br = 1024
    full = lambda: pl.BlockSpec((128, 128), lambda i: (0, 0))
    bias = lambda: pl.BlockSpec((1, 128), lambda i: (0, 0))
    rowb = lambda: pl.BlockSpec((br, 128), lambda i: (i, 0))

    def body(s_ref, g_ref, w1, b1, w2, b2, v1, c1, v2, c2, o_ref):
        x = s_ref[...]
        h = _elu(jnp.dot(x, w1[...], preferred_element_type=jnp.float32) + b1[...])
        h = jnp.dot(h, w2[...], preferred_element_type=jnp.float32) + b2[...] + g_ref[...]
        h2 = _elu(jnp.dot(h, v1[...], preferred_element_type=jnp.float32) + c1[...])
        o_ref[...] = jnp.dot(h2, v2[...], preferred_element_type=jnp.float32) + c2[...]

    return pl.pallas_call(
        body,
        grid=(mp // br,),
        in_specs=[rowb(), rowb(), full(), bias(), full(), bias(), full(), bias(), full(), bias()],
        out_specs=rowb(),
        out_shape=jax.ShapeDtypeStruct((mp, 128), jnp.float32),
    )(snf128, g128, w1d, b1t, w2d, b2t, v1d, c1t, v2d, c2t)


def _blkdiag4(w):
    z = jnp.zeros((32, 32), w.dtype)
    return jnp.block([[w, z, z, z], [z, w, z, z], [z, z, w, z], [z, z, z, w]])


BR = 512


def _tc_row_mlps_segsum(snf, g, seg, nn_W1, nn_b1, nn_W2, nn_b2,
                        gm1_W1, gm1_b1, gm1_W2, gm1_b2):
    m = snf.shape[0]
    nb = m // BR
    firsts = seg[::BR]
    seg3 = seg.reshape(nb, 1, BR)
    full = lambda: pl.BlockSpec((32, 32), lambda i, f: (0, 0))
    bias = lambda: pl.BlockSpec((1, 32), lambda i, f: (0, 0))
    rowb = lambda: pl.BlockSpec((BR, 32), lambda i, f: (i, 0))

    def body(f_ref, s_ref, g_ref, seg_ref, w1, b1, w2, b2, v1, c1, v2, c2, o_ref):
        i = pl.program_id(0)

        @pl.when(i == 0)
        def _init():
            o_ref[...] = jnp.zeros((NSEG + BR, 32), jnp.float32)

        x = s_ref[...]
        h = _elu(jnp.dot(x, w1[...], preferred_element_type=jnp.float32) + b1[...])
        h = jnp.dot(h, w2[...], preferred_element_type=jnp.float32) + b2[...] + g_ref[...]
        h2 = _elu(jnp.dot(h, v1[...], preferred_element_type=jnp.float32) + c1[...])
        y = jnp.dot(h2, v2[...], preferred_element_type=jnp.float32) + c2[...]

        first = f_ref[i]
        rel = seg_ref[0] - first
        pt = (jax.lax.broadcasted_iota(jnp.int32, (BR, BR), 0) == rel
              ).astype(jnp.bfloat16)
        yh = y.astype(jnp.bfloat16)
        yl = (y - yh.astype(jnp.float32)).astype(jnp.bfloat16)
        sblk = (jnp.dot(pt, yh, preferred_element_type=jnp.float32)
                + jnp.dot(pt, yl, preferred_element_type=jnp.float32))
        o_ref[pl.ds(first, BR), :] += sblk

    acc = pl.pallas_call(
        body,
        grid_spec=pltpu.PrefetchScalarGridSpec(
            num_scalar_prefetch=1,
            grid=(nb,),
            in_specs=[rowb(), rowb(), pl.BlockSpec((1, 1, BR), lambda i, f: (i, 0, 0)),
                      full(), bias(), full(), bias(), full(), bias(), full(), bias()],
            out_specs=pl.BlockSpec((NSEG + BR, 32), lambda i, f: (0, 0)),
        ),
        out_shape=jax.ShapeDtypeStruct((NSEG + BR, 32), jnp.float32),
    )(firsts, snf, g, seg3, nn_W1, nn_b1.reshape(1, 32), nn_W2, nn_b2.reshape(1, 32),
      gm1_W1, gm1_b1.reshape(1, 32), gm1_W2, gm1_b2.reshape(1, 32))
    return acc[:NSEG]


def _tc_head(s1p, s2p, cp, gm2_W1, gm2_b1, gm2_W2, gm2_b2,
             fm_W1, fm_b1, fm_W2, fm_b2, oW1f, oW1g, ob1, oW2, ob2):
    br = 2048
    n_label = oW2.shape[1]
    full = lambda: pl.BlockSpec((32, 32), lambda i: (0, 0))
    bias = lambda: pl.BlockSpec((1, 32), lambda i: (0, 0))

    def body(s1_ref, s2_ref, c_ref, g1, gb1, g2, gb2, f1, fb1, f2, fb2,
             w1f, w1g, b1, w2, b2, o_ref):
        s1 = s1_ref[0] + s1_ref[1]
        s2 = s2_ref[0] + s2_ref[1]
        cnt = c_ref[0] + c_ref[1]
        cnt1 = jnp.clip(cnt[:, 0:1], 1.0, None)
        og = _elu(jnp.dot(s1, g1[...], preferred_element_type=jnp.float32) + gb1[...])
        og = jnp.dot(og, g2[...], preferred_element_type=jnp.float32) + gb2[...]
        mean = s2 / cnt1
        of = _elu(jnp.dot(mean, f1[...], preferred_element_type=jnp.float32) + fb1[...])
        of = jnp.dot(of, f2[...], preferred_element_type=jnp.float32) + fb2[...]
        h = _elu(jnp.dot(of, w1f[...], preferred_element_type=jnp.float32)
                 + jnp.dot(og, w1g[...], preferred_element_type=jnp.float32) + b1[...])
        o_ref[...] = jnp.dot(h, w2[...], preferred_element_type=jnp.float32) + b2[...]

    return pl.pallas_call(
        body,
        grid=(NSEG // br,),
        in_specs=[
            pl.BlockSpec((2, br, 32), lambda i: (0, i, 0)),
            pl.BlockSpec((2, br, 32), lambda i: (0, i, 0)),
            pl.BlockSpec((2, br, 16), lambda i: (0, i, 0)),
            full(), bias(), full(), bias(), full(), bias(), full(), bias(),
            full(), full(), bias(),
            pl.BlockSpec((32, n_label), lambda i: (0, 0)),
            pl.BlockSpec((1, n_label), lambda i: (0, 0)),
        ],
        out_specs=pl.BlockSpec((br, n_label), lambda i: (i, 0)),
        out_shape=jax.ShapeDtypeStruct((NSEG, n_label), jnp.float32),
    )(s1p, s2p, cp, gm2_W1, gm2_b1.reshape(1, 32), gm2_W2, gm2_b2.reshape(1, 32),
      fm_W1, fm_b1.reshape(1, 32), fm_W2, fm_b2.reshape(1, 32),
      oW1f, oW1g, ob1.reshape(1, 32), oW2, ob2.reshape(1, n_label))


def kernel(graph_out, sample_node_id, sample_node_feature, sample_id, sample_feature,
           nn_W1, nn_b1, nn_W2, nn_b2, gm1_W1, gm1_b1, gm1_W2, gm1_b2,
           gm2_W1, gm2_b1, gm2_W2, gm2_b2, fm_W1, fm_b1, fm_W2, fm_b2,
           out_W1, out_b1, out_W2, out_b2):
    seg1 = sample_node_id[:, 0]
    nid = sample_node_id[:, 1]

    m = sample_node_feature.shape[0]
    r = sample_feature.shape[0]
    g128 = _sc_gather(graph_out, nid.reshape(m // 128, 128))
    s2p, cp = _sc_segsum2(sample_feature, sample_id.reshape(r // 128, 128))
    y128 = _tc_row_mlps_packed(
        sample_node_feature.reshape(m // 4, 128), g128,
        _blkdiag4(nn_W1), jnp.tile(nn_b1, 4).reshape(1, 128),
        _blkdiag4(nn_W2), jnp.tile(nn_b2, 4).reshape(1, 128),
        _blkdiag4(gm1_W1), jnp.tile(gm1_b1, 4).reshape(1, 128),
        _blkdiag4(gm1_W2), jnp.tile(gm1_b2, 4).reshape(1, 128))
    s1p = _sc_segsum1(y128.reshape(m, 32), seg1.reshape(m // 128, 128))
    return _tc_head(s1p, s2p, cp, gm2_W1, gm2_b1, gm2_W2, gm2_b2,
                    fm_W1, fm_b1, fm_W2, fm_b2,
                    out_W1[:32], out_W1[32:], out_b1, out_W2, out_b2)
